# baseline (speedup 1.0000x reference)
import jax
import jax.numpy as jnp
import numpy as np
from jax.experimental import pallas as pl
from jax.experimental.pallas import tpu as pltpu

SLEN = 100
PTILE = 8
STEP = 2
NH = (SLEN - PTILE) // STEP + 1
B = 8
CC = 10
PIX = PTILE * PTILE
FIN = CC * PIX
DIM_OUT = 69


def _tap_tensor():
    e = np.zeros((3, PTILE, PTILE), np.float32)
    for k in range(3):
        for o in range(PTILE):
            i = o + k - 1
            if 0 <= i < PTILE:
                e[k, i, o] = 1.0
    ee = (e[:, None, :, None, :, None] * e[None, :, None, :, None, :])
    return ee.reshape(9, PIX, PIX)


_PERM = np.array([y * PTILE + 2 * j + s
                  for s in range(2) for j in range(PTILE // 2)
                  for y in range(PTILE)])
_EE = _tap_tensor()


def _conv_as_dense(conv1_w, conv2_w):
    ee1 = jnp.asarray(_EE[:, _PERM, :].reshape(9, PIX * PIX))
    p1 = conv1_w.reshape(CC, 9) @ ee1
    m1 = p1.reshape(CC, PIX, PIX).transpose(1, 0, 2).reshape(PIX, FIN)
    ee2 = jnp.asarray(_EE.reshape(9, PIX * PIX))
    w2f = conv2_w.transpose(1, 0, 2, 3).reshape(CC * CC, 9)
    p2 = w2f @ ee2
    m2 = p2.reshape(CC, CC, PIX, PIX).transpose(0, 2, 1, 3).reshape(FIN, FIN)
    return m1, m2


def _fused(ime_ref, imo_ref, m1_ref, b1_ref, m2_ref, b2_ref, w3_ref, b3_ref,
           w4_ref, b4_ref, w5_ref, b5_ref, w6_ref, b6_ref, out_ref):
    ih = pl.program_id(0)
    re = ime_ref[:, pl.ds(ih * STEP, PTILE), :]
    ro = imo_ref[:, pl.ds(ih * STEP, PTILE), :]
    parts = [src[:, :, j: j + NH] for src in (re, ro) for j in range(PTILE // 2)]
    t = jnp.concatenate(parts, axis=1)
    h = jax.lax.dot_general(t.astype(jnp.bfloat16), m1_ref[...],
                            (((1,), (0,)), ((), ())),
                            preferred_element_type=jnp.float32)
    h = jnp.maximum(h.reshape(B * NH, FIN) + b1_ref[...], 0.0)
    h = jnp.maximum(jnp.dot(h.astype(jnp.bfloat16), m2_ref[...], preferred_element_type=jnp.float32) + b2_ref[...], 0.0)
    h = jnp.maximum(jnp.dot(h.astype(jnp.bfloat16), w3_ref[...], preferred_element_type=jnp.float32) + b3_ref[...], 0.0)
    h = jnp.maximum(jnp.dot(h.astype(jnp.bfloat16), w4_ref[...], preferred_element_type=jnp.float32) + b4_ref[...], 0.0)
    h = jnp.maximum(jnp.dot(h.astype(jnp.bfloat16), w5_ref[...], preferred_element_type=jnp.float32) + b5_ref[...], 0.0)
    h = jnp.dot(h.astype(jnp.bfloat16), w6_ref[...], preferred_element_type=jnp.float32) + b6_ref[...]
    out_ref[...] = jnp.transpose(h.reshape(B, NH, DIM_OUT), (1, 0, 2))


def kernel(images, conv1_w, conv1_b, conv2_w, conv2_b, fc1_w, fc1_b,
           fc2_w, fc2_b, fc3_w, fc3_b, fcf_w, fcf_b):
    im = images[:, 0]
    ime = im[:, :, 0::2]
    imo = im[:, :, 1::2]
    m1, m2 = _conv_as_dense(conv1_w, conv2_w)
    m1 = m1.astype(jnp.bfloat16)
    m2 = m2.astype(jnp.bfloat16)
    b1 = jnp.repeat(conv1_b, PIX).reshape(1, FIN)
    b2 = jnp.repeat(conv2_b, PIX).reshape(1, FIN)
    full = lambda shape: pl.BlockSpec(shape, lambda i: (0,) * len(shape))
    out = pl.pallas_call(
        _fused,
        grid=(NH,),
        in_specs=[
            full((B, SLEN, SLEN // 2)), full((B, SLEN, SLEN // 2)),
            full((PIX, FIN)), full((1, FIN)),
            full((FIN, FIN)), full((1, FIN)),
            full((FIN, 64)), full((1, 64)),
            full((64, 64)), full((1, 64)),
            full((64, 64)), full((1, 64)),
            full((64, DIM_OUT)), full((1, DIM_OUT)),
        ],
        out_specs=pl.BlockSpec((NH, B, DIM_OUT), lambda i: (i, 0, 0)),
        out_shape=jax.ShapeDtypeStruct((NH * NH, B, DIM_OUT), jnp.float32),
        compiler_params=pltpu.CompilerParams(dimension_semantics=("arbitrary",)),
    )(ime, imo, m1, b1, m2, b2,
      fc1_w.T.astype(jnp.bfloat16), fc1_b.reshape(1, 64),
      fc2_w.T.astype(jnp.bfloat16), fc2_b.reshape(1, 64),
      fc3_w.T.astype(jnp.bfloat16), fc3_b.reshape(1, 64),
      fcf_w.T.astype(jnp.bfloat16), fcf_b.reshape(1, DIM_OUT))
    return out.reshape(NH * NH * B, DIM_OUT)


# speedup vs baseline: 1.0006x; 1.0006x over previous
import jax
import jax.numpy as jnp
import numpy as np
from jax.experimental import pallas as pl
from jax.experimental.pallas import tpu as pltpu

SLEN = 100
PTILE = 8
STEP = 2
NH = (SLEN - PTILE) // STEP + 1
B = 8
CC = 10
PIX = PTILE * PTILE
FIN = CC * PIX
DIM_OUT = 69


def _tap_tensor():
    e = np.zeros((3, PTILE, PTILE), np.float32)
    for k in range(3):
        for o in range(PTILE):
            i = o + k - 1
            if 0 <= i < PTILE:
                e[k, i, o] = 1.0
    ee = (e[:, None, :, None, :, None] * e[None, :, None, :, None, :])
    return ee.reshape(9, PIX, PIX)


_PERM = np.array([y * PTILE + 2 * j + s
                  for s in range(2) for j in range(PTILE // 2)
                  for y in range(PTILE)])
_EE = _tap_tensor()


def _conv_as_dense(conv1_w, conv2_w):
    ee1 = jnp.asarray(_EE[:, _PERM, :])
    w1f = conv1_w.reshape(CC, 9)
    m1 = (ee1[:, :, None, :] * w1f.T[:, None, :, None]).sum(0).reshape(PIX, FIN)
    ee2 = jnp.asarray(_EE)
    w2f = conv2_w.reshape(CC, CC, 9)
    m2 = (w2f.transpose(2, 1, 0)[:, :, None, :, None]
          * ee2[:, None, :, None, :]).sum(0).reshape(FIN, FIN)
    return m1, m2


def _fused(ime_ref, imo_ref, m1_ref, b1_ref, m2_ref, b2_ref, w3_ref, b3_ref,
           w4_ref, b4_ref, w5_ref, b5_ref, w6_ref, b6_ref, out_ref):
    ih = pl.program_id(0)
    re = ime_ref[:, pl.ds(ih * STEP, PTILE), :]
    ro = imo_ref[:, pl.ds(ih * STEP, PTILE), :]
    parts = [src[:, :, j: j + NH] for src in (re, ro) for j in range(PTILE // 2)]
    t = jnp.concatenate(parts, axis=1)
    h = jax.lax.dot_general(t.astype(jnp.bfloat16), m1_ref[...],
                            (((1,), (0,)), ((), ())),
                            preferred_element_type=jnp.float32)
    h = jnp.maximum(h.reshape(B * NH, FIN) + b1_ref[...], 0.0)
    h = jnp.maximum(jnp.dot(h.astype(jnp.bfloat16), m2_ref[...], preferred_element_type=jnp.float32) + b2_ref[...], 0.0)
    h = jnp.maximum(jnp.dot(h.astype(jnp.bfloat16), w3_ref[...], preferred_element_type=jnp.float32) + b3_ref[...], 0.0)
    h = jnp.maximum(jnp.dot(h.astype(jnp.bfloat16), w4_ref[...], preferred_element_type=jnp.float32) + b4_ref[...], 0.0)
    h = jnp.maximum(jnp.dot(h.astype(jnp.bfloat16), w5_ref[...], preferred_element_type=jnp.float32) + b5_ref[...], 0.0)
    h = jnp.dot(h.astype(jnp.bfloat16), w6_ref[...], preferred_element_type=jnp.float32) + b6_ref[...]
    out_ref[...] = jnp.transpose(h.reshape(B, NH, DIM_OUT), (1, 0, 2))


def kernel(images, conv1_w, conv1_b, conv2_w, conv2_b, fc1_w, fc1_b,
           fc2_w, fc2_b, fc3_w, fc3_b, fcf_w, fcf_b):
    im = images[:, 0]
    ime = im[:, :, 0::2]
    imo = im[:, :, 1::2]
    m1, m2 = _conv_as_dense(conv1_w, conv2_w)
    m1 = m1.astype(jnp.bfloat16)
    m2 = m2.astype(jnp.bfloat16)
    b1 = jnp.repeat(conv1_b, PIX).reshape(1, FIN)
    b2 = jnp.repeat(conv2_b, PIX).reshape(1, FIN)
    full = lambda shape: pl.BlockSpec(shape, lambda i: (0,) * len(shape))
    out = pl.pallas_call(
        _fused,
        grid=(NH,),
        in_specs=[
            full((B, SLEN, SLEN // 2)), full((B, SLEN, SLEN // 2)),
            full((PIX, FIN)), full((1, FIN)),
            full((FIN, FIN)), full((1, FIN)),
            full((FIN, 64)), full((1, 64)),
            full((64, 64)), full((1, 64)),
            full((64, 64)), full((1, 64)),
            full((64, DIM_OUT)), full((1, DIM_OUT)),
        ],
        out_specs=pl.BlockSpec((NH, B, DIM_OUT), lambda i: (i, 0, 0)),
        out_shape=jax.ShapeDtypeStruct((NH * NH, B, DIM_OUT), jnp.float32),
        compiler_params=pltpu.CompilerParams(dimension_semantics=("arbitrary",)),
    )(ime, imo, m1, b1, m2, b2,
      fc1_w.T.astype(jnp.bfloat16), fc1_b.reshape(1, 64),
      fc2_w.T.astype(jnp.bfloat16), fc2_b.reshape(1, 64),
      fc3_w.T.astype(jnp.bfloat16), fc3_b.reshape(1, 64),
      fcf_w.T.astype(jnp.bfloat16), fcf_b.reshape(1, DIM_OUT))
    return out.reshape(NH * NH * B, DIM_OUT)
